# Initial kernel scaffold; baseline (speedup 1.0000x reference)
import jax
import jax.numpy as jnp
from jax.experimental import pallas as pl


def kernel(item_input, enc_W1, enc_b1, enc_W2, enc_b2, lstm_kernel, lstm_bias, dec_W1, dec_b1, dec_W2, dec_b2, dec_W3, dec_b3):
    raise NotImplementedError("write your pallas kernel here")


# speedup vs baseline: 1.3241x; 1.3241x over previous
import jax
import jax.numpy as jnp
from jax.experimental import pallas as pl
from jax.experimental.pallas import tpu as pltpu

S, F, H = 50, 128, 32
BB = 128


def _decode_block(item_ref, g_ref, encW1_ref, encb1_ref, encW2_ref, encb2_ref,
                  Wx_ref, Wh_ref, lb_ref, dW1e_ref, dW1h_ref, db1_ref,
                  dW2_ref, db2_ref, w3t_ref, db3_ref,
                  probs_all_ref, probs_one_ref, idx_ref, scores_ref):
    item = item_ref[...]
    item2 = item.reshape(BB * S, F)

    e1 = jnp.maximum(
        jnp.dot(item2, encW1_ref[...], preferred_element_type=jnp.float32)
        + encb1_ref[...], 0.0)
    e2 = jnp.maximum(
        jnp.dot(e1, encW2_ref[...], preferred_element_type=jnp.float32)
        + encb2_ref[...], 0.0)
    encproj = (jnp.dot(e2, dW1e_ref[...], preferred_element_type=jnp.float32)
               + db1_ref[...]).reshape(BB, S, 32)

    iota2 = jax.lax.broadcasted_iota(jnp.int32, (BB, S), 1)
    iota3 = jax.lax.broadcasted_iota(jnp.int32, (BB, S, F), 1)

    def body(k, carry):
        h, c, mask, dec_in, p1, ia, sa = carry
        gates = (jnp.dot(dec_in, Wx_ref[...], preferred_element_type=jnp.float32)
                 + jnp.dot(h, Wh_ref[...], preferred_element_type=jnp.float32)
                 + lb_ref[...])
        gi = gates[:, 0:H]
        gj = gates[:, H:2 * H]
        gf = gates[:, 2 * H:3 * H]
        go = gates[:, 3 * H:4 * H]
        c = c * jax.nn.sigmoid(gf + 1.0) + jax.nn.sigmoid(gi) * jnp.tanh(gj)
        h = jnp.tanh(c) * jax.nn.sigmoid(go)

        hp = jnp.dot(h, dW1h_ref[...], preferred_element_type=jnp.float32)
        z1 = jnp.maximum(encproj + hp[:, None, :], 0.0)
        z2 = jnp.maximum(
            jnp.dot(z1.reshape(BB * S, 32), dW2_ref[...],
                    preferred_element_type=jnp.float32) + db2_ref[...], 0.0)
        z2 = z2.reshape(BB, S, 16)
        logits = jnp.sum(z2 * w3t_ref[...][None], axis=2) + db3_ref[...]

        ml = (1.0 - mask) * (-1e9) + logits
        m = jnp.max(ml, axis=1, keepdims=True)
        e = jnp.exp(ml - m)
        p = e / jnp.sum(e, axis=1, keepdims=True)

        y = jnp.log(p + 1e-20) + g_ref[k]
        mx = jnp.max(y, axis=1, keepdims=True)
        idxv = jnp.min(jnp.where(y == mx, iota2, S), axis=1, keepdims=True)
        oh = (iota2 == idxv).astype(jnp.float32)

        mask = mask - oh
        p1 = jnp.where(iota2 == k, jnp.sum(p * oh, axis=1, keepdims=True), p1)
        ia = jnp.where(iota2 == k, idxv, ia)
        sa = sa + oh * (1.0 - 0.03 * k.astype(jnp.float32))

        probs_all_ref[:, pl.ds(k, 1), :] = p[:, None, :]

        dec_in = jnp.sum(jnp.where(iota3 == idxv[:, :, None], item, 0.0), axis=1)
        return h, c, mask, dec_in, p1, ia, sa

    zf = jnp.zeros((BB, H), jnp.float32)
    carry = (zf, zf, jnp.ones((BB, S), jnp.float32), jnp.zeros((BB, F), jnp.float32),
             jnp.zeros((BB, S), jnp.float32), jnp.zeros((BB, S), jnp.int32),
             jnp.zeros((BB, S), jnp.float32))
    h, c, mask, dec_in, p1, ia, sa = jax.lax.fori_loop(0, S, body, carry)
    probs_one_ref[...] = p1
    idx_ref[...] = ia
    scores_ref[...] = sa


def kernel(item_input, enc_W1, enc_b1, enc_W2, enc_b2, lstm_kernel, lstm_bias,
           dec_W1, dec_b1, dec_W2, dec_b2, dec_W3, dec_b3):
    b = item_input.shape[0]
    nb = b // BB

    base = jax.random.key(42)
    g = jnp.stack([jax.random.gumbel(jax.random.fold_in(base, k), (b, S), jnp.float32)
                   for k in range(S)])

    Wx = lstm_kernel[:F]
    Wh = lstm_kernel[F:]
    dW1e = dec_W1[:16]
    dW1h = dec_W1[16:]

    full = lambda shp: pl.BlockSpec(shp, lambda i: tuple(0 for _ in shp))
    in_specs = [
        pl.BlockSpec((BB, S, F), lambda i: (i, 0, 0)),
        pl.BlockSpec((S, BB, S), lambda i: (0, i, 0)),
        full((F, 32)), full((1, 32)), full((32, 16)), full((1, 16)),
        full((F, 4 * H)), full((H, 4 * H)), full((1, 4 * H)),
        full((16, 32)), full((H, 32)), full((1, 32)),
        full((32, 16)), full((1, 16)), full((1, 16)), full((1, 1)),
    ]
    out_specs = [
        pl.BlockSpec((BB, S, S), lambda i: (i, 0, 0)),
        pl.BlockSpec((BB, S), lambda i: (i, 0)),
        pl.BlockSpec((BB, S), lambda i: (i, 0)),
        pl.BlockSpec((BB, S), lambda i: (i, 0)),
    ]
    out_shape = [
        jax.ShapeDtypeStruct((b, S, S), jnp.float32),
        jax.ShapeDtypeStruct((b, S), jnp.float32),
        jax.ShapeDtypeStruct((b, S), jnp.int32),
        jax.ShapeDtypeStruct((b, S), jnp.float32),
    ]
    probs_all, probs_one, idx, scores = pl.pallas_call(
        _decode_block,
        grid=(nb,),
        in_specs=in_specs,
        out_specs=out_specs,
        out_shape=out_shape,
        compiler_params=pltpu.CompilerParams(
            dimension_semantics=("parallel",)),
    )(item_input, g, enc_W1, enc_b1.reshape(1, 32), enc_W2, enc_b2.reshape(1, 16),
      Wx, Wh, lstm_bias.reshape(1, 4 * H), dW1e, dW1h, dec_b1.reshape(1, 32),
      dec_W2, dec_b2.reshape(1, 16),
      dec_W3.reshape(1, 16), dec_b3.reshape(1, 1))
    return (probs_all, probs_one, idx, scores.reshape(-1, 1))


# speedup vs baseline: 2.4872x; 1.8784x over previous
import jax
import jax.numpy as jnp
from jax.experimental import pallas as pl
from jax.experimental.pallas import tpu as pltpu

S, F, H = 50, 128, 32
BB = 128


def _decode_block(item_ref, g_ref, encW1_ref, encb1_ref, encW2_ref, encb2_ref,
                  Wx_ref, Wh_ref, lb_ref, dW1e_ref, dW1h_ref, db1_ref,
                  dW2_ref, db2_ref, w3c_ref, db3_ref,
                  probs_all_ref, probs_one_ref, idx_ref, scores_ref):
    item2 = item_ref[...].reshape(S * BB, F)

    e1 = jnp.maximum(
        jnp.dot(item2, encW1_ref[...], preferred_element_type=jnp.float32)
        + encb1_ref[...], 0.0)
    e2 = jnp.maximum(
        jnp.dot(e1, encW2_ref[...], preferred_element_type=jnp.float32)
        + encb2_ref[...], 0.0)
    encproj = (jnp.dot(e2, dW1e_ref[...], preferred_element_type=jnp.float32)
               + db1_ref[...]).reshape(S, BB, 32)

    iota_sb = jax.lax.broadcasted_iota(jnp.int32, (S, BB), 0)
    iota_l = jax.lax.broadcasted_iota(jnp.int32, (S, BB, 1), 0)

    def body(k, carry):
        h, c, mask, dec_in, p1, ia, sa = carry
        gates = (jnp.dot(dec_in, Wx_ref[...], preferred_element_type=jnp.float32)
                 + jnp.dot(h, Wh_ref[...], preferred_element_type=jnp.float32)
                 + lb_ref[...])
        gi = gates[:, 0:H]
        gj = gates[:, H:2 * H]
        gf = gates[:, 2 * H:3 * H]
        go = gates[:, 3 * H:4 * H]
        c = c * jax.nn.sigmoid(gf + 1.0) + jax.nn.sigmoid(gi) * jnp.tanh(gj)
        h = jnp.tanh(c) * jax.nn.sigmoid(go)

        hp = jnp.dot(h, dW1h_ref[...], preferred_element_type=jnp.float32)
        z1 = jnp.maximum(encproj + hp[None, :, :], 0.0)
        z2 = jnp.maximum(
            jnp.dot(z1.reshape(S * BB, 32), dW2_ref[...],
                    preferred_element_type=jnp.float32) + db2_ref[...], 0.0)
        lcol = jnp.dot(z2, w3c_ref[...],
                       preferred_element_type=jnp.float32)
        logits = jnp.transpose(lcol.reshape(S, BB, 1), (0, 2, 1))[:, 0, :]
        logits = logits + db3_ref[...]

        ml = (1.0 - mask) * (-1e9) + logits
        m = jnp.max(ml, axis=0, keepdims=True)
        e = jnp.exp(ml - m)
        p = e / jnp.sum(e, axis=0, keepdims=True)

        y = jnp.log(p + 1e-20) + g_ref[pl.ds(k, 1)][0]
        mx = jnp.max(y, axis=0, keepdims=True)
        idxv = jnp.min(jnp.where(y == mx, iota_sb, S), axis=0, keepdims=True)
        oh = (iota_sb == idxv).astype(jnp.float32)

        mask = mask - oh
        p1 = jnp.where(iota_sb == k, jnp.sum(p * oh, axis=0, keepdims=True), p1)
        ia = jnp.where(iota_sb == k, idxv, ia)
        sa = sa + oh * (1.0 - 0.03 * k.astype(jnp.float32))

        probs_all_ref[pl.ds(k, 1)] = p[None]

        idx_col = jnp.transpose(idxv)
        sel = iota_l == idx_col[None]
        dec_in = jnp.sum(jnp.where(sel, item_ref[...], 0.0), axis=0)
        return h, c, mask, dec_in, p1, ia, sa

    zf = jnp.zeros((BB, H), jnp.float32)
    carry = (zf, zf, jnp.ones((S, BB), jnp.float32), jnp.zeros((BB, F), jnp.float32),
             jnp.zeros((S, BB), jnp.float32), jnp.zeros((S, BB), jnp.int32),
             jnp.zeros((S, BB), jnp.float32))
    h, c, mask, dec_in, p1, ia, sa = jax.lax.fori_loop(0, S, body, carry)
    probs_one_ref[...] = p1
    idx_ref[...] = ia
    scores_ref[...] = sa


def kernel(item_input, enc_W1, enc_b1, enc_W2, enc_b2, lstm_kernel, lstm_bias,
           dec_W1, dec_b1, dec_W2, dec_b2, dec_W3, dec_b3):
    b = item_input.shape[0]
    nb = b // BB

    base = jax.random.key(42)
    g = jnp.stack([jax.random.gumbel(jax.random.fold_in(base, k), (b, S), jnp.float32)
                   for k in range(S)])
    g_t = jnp.transpose(g, (0, 2, 1))
    item_t = jnp.transpose(item_input, (1, 0, 2))

    Wx = lstm_kernel[:F]
    Wh = lstm_kernel[F:]
    dW1e = dec_W1[:16]
    dW1h = dec_W1[16:]

    full = lambda shp: pl.BlockSpec(shp, lambda i: tuple(0 for _ in shp))
    in_specs = [
        pl.BlockSpec((S, BB, F), lambda i: (0, i, 0)),
        pl.BlockSpec((S, S, BB), lambda i: (0, 0, i)),
        full((F, 32)), full((1, 32)), full((32, 16)), full((1, 16)),
        full((F, 4 * H)), full((H, 4 * H)), full((1, 4 * H)),
        full((16, 32)), full((H, 32)), full((1, 32)),
        full((32, 16)), full((1, 16)), full((16, 1)), full((1, 1)),
    ]
    out_specs = [
        pl.BlockSpec((S, S, BB), lambda i: (0, 0, i)),
        pl.BlockSpec((S, BB), lambda i: (0, i)),
        pl.BlockSpec((S, BB), lambda i: (0, i)),
        pl.BlockSpec((S, BB), lambda i: (0, i)),
    ]
    out_shape = [
        jax.ShapeDtypeStruct((S, S, b), jnp.float32),
        jax.ShapeDtypeStruct((S, b), jnp.float32),
        jax.ShapeDtypeStruct((S, b), jnp.int32),
        jax.ShapeDtypeStruct((S, b), jnp.float32),
    ]
    pa_t, p1_t, idx_t, sc_t = pl.pallas_call(
        _decode_block,
        grid=(nb,),
        in_specs=in_specs,
        out_specs=out_specs,
        out_shape=out_shape,
        compiler_params=pltpu.CompilerParams(
            dimension_semantics=("parallel",)),
    )(item_t, g_t, enc_W1, enc_b1.reshape(1, 32), enc_W2, enc_b2.reshape(1, 16),
      Wx, Wh, lstm_bias.reshape(1, 4 * H), dW1e, dW1h, dec_b1.reshape(1, 32),
      dec_W2, dec_b2.reshape(1, 16),
      dec_W3, dec_b3.reshape(1, 1))
    probs_all = jnp.transpose(pa_t, (2, 0, 1))
    probs_one = jnp.transpose(p1_t)
    idx = jnp.transpose(idx_t)
    scores = jnp.transpose(sc_t)
    return (probs_all, probs_one, idx, scores.reshape(-1, 1))


# speedup vs baseline: 2.8663x; 1.1524x over previous
import jax
import jax.numpy as jnp
from jax.experimental import pallas as pl
from jax.experimental.pallas import tpu as pltpu

S, F, H = 50, 128, 32
BB = 256


def _decode_block(item_ref, g_ref, encW1_ref, encb1_ref, encW2_ref, encb2_ref,
                  Wx_ref, Wh_ref, lb_ref, dW1e_ref, dW1h_ref, db1_ref,
                  dW2_ref, db2_ref, w3c_ref, db3_ref,
                  probs_all_ref, probs_one_ref, idx_ref, scores_ref):
    item2 = item_ref[...].reshape(S * BB, F)

    e1 = jnp.maximum(
        jnp.dot(item2, encW1_ref[...], preferred_element_type=jnp.float32)
        + encb1_ref[...], 0.0)
    e2 = jnp.maximum(
        jnp.dot(e1, encW2_ref[...], preferred_element_type=jnp.float32)
        + encb2_ref[...], 0.0)
    encproj = (jnp.dot(e2, dW1e_ref[...], preferred_element_type=jnp.float32)
               + db1_ref[...]).reshape(S, BB, 32)

    iota_sb = jax.lax.broadcasted_iota(jnp.int32, (S, BB), 0)
    iota_l = jax.lax.broadcasted_iota(jnp.int32, (S, BB, 1), 0)

    def body(k, carry):
        h, c, mask, dec_in, p1, ia, sa = carry
        gates = (jnp.dot(dec_in, Wx_ref[...], preferred_element_type=jnp.float32)
                 + jnp.dot(h, Wh_ref[...], preferred_element_type=jnp.float32)
                 + lb_ref[...])
        gi = gates[:, 0:H]
        gj = gates[:, H:2 * H]
        gf = gates[:, 2 * H:3 * H]
        go = gates[:, 3 * H:4 * H]
        c = c * jax.nn.sigmoid(gf + 1.0) + jax.nn.sigmoid(gi) * jnp.tanh(gj)
        h = jnp.tanh(c) * jax.nn.sigmoid(go)

        hp = jnp.dot(h, dW1h_ref[...], preferred_element_type=jnp.float32)
        z1 = jnp.maximum(encproj + hp[None, :, :], 0.0)
        z2 = jnp.maximum(
            jnp.dot(z1.reshape(S * BB, 32), dW2_ref[...],
                    preferred_element_type=jnp.float32) + db2_ref[...], 0.0)
        lcol = jnp.dot(z2, w3c_ref[...],
                       preferred_element_type=jnp.float32)
        logits = jnp.transpose(lcol.reshape(S, BB, 1), (0, 2, 1))[:, 0, :]
        logits = logits + db3_ref[...]

        ml = (1.0 - mask) * (-1e9) + logits
        m = jnp.max(ml, axis=0, keepdims=True)
        e = jnp.exp(ml - m)
        p = e / jnp.sum(e, axis=0, keepdims=True)

        y = jnp.log(p + 1e-20) + g_ref[pl.ds(k, 1)][0]
        mx = jnp.max(y, axis=0, keepdims=True)
        idxv = jnp.min(jnp.where(y == mx, iota_sb, S), axis=0, keepdims=True)
        oh = (iota_sb == idxv).astype(jnp.float32)

        mask = mask - oh
        p1 = jnp.where(iota_sb == k, jnp.sum(p * oh, axis=0, keepdims=True), p1)
        ia = jnp.where(iota_sb == k, idxv, ia)
        sa = sa + oh * (1.0 - 0.03 * k.astype(jnp.float32))

        probs_all_ref[pl.ds(k, 1)] = p[None]

        idx_col = jnp.transpose(idxv)
        sel = iota_l == idx_col[None]
        dec_in = jnp.sum(jnp.where(sel, item_ref[...], 0.0), axis=0)
        return h, c, mask, dec_in, p1, ia, sa

    zf = jnp.zeros((BB, H), jnp.float32)
    carry = (zf, zf, jnp.ones((S, BB), jnp.float32), jnp.zeros((BB, F), jnp.float32),
             jnp.zeros((S, BB), jnp.float32), jnp.zeros((S, BB), jnp.int32),
             jnp.zeros((S, BB), jnp.float32))
    h, c, mask, dec_in, p1, ia, sa = jax.lax.fori_loop(0, S, body, carry)
    probs_one_ref[...] = p1
    idx_ref[...] = ia
    scores_ref[...] = sa


def kernel(item_input, enc_W1, enc_b1, enc_W2, enc_b2, lstm_kernel, lstm_bias,
           dec_W1, dec_b1, dec_W2, dec_b2, dec_W3, dec_b3):
    b = item_input.shape[0]
    nb = b // BB

    base = jax.random.key(42)
    g = jnp.stack([jax.random.gumbel(jax.random.fold_in(base, k), (b, S), jnp.float32)
                   for k in range(S)])
    g_t = jnp.transpose(g, (0, 2, 1))
    item_t = jnp.transpose(item_input, (1, 0, 2))

    Wx = lstm_kernel[:F]
    Wh = lstm_kernel[F:]
    dW1e = dec_W1[:16]
    dW1h = dec_W1[16:]

    full = lambda shp: pl.BlockSpec(shp, lambda i: tuple(0 for _ in shp))
    in_specs = [
        pl.BlockSpec((S, BB, F), lambda i: (0, i, 0)),
        pl.BlockSpec((S, S, BB), lambda i: (0, 0, i)),
        full((F, 32)), full((1, 32)), full((32, 16)), full((1, 16)),
        full((F, 4 * H)), full((H, 4 * H)), full((1, 4 * H)),
        full((16, 32)), full((H, 32)), full((1, 32)),
        full((32, 16)), full((1, 16)), full((16, 1)), full((1, 1)),
    ]
    out_specs = [
        pl.BlockSpec((S, S, BB), lambda i: (0, 0, i)),
        pl.BlockSpec((S, BB), lambda i: (0, i)),
        pl.BlockSpec((S, BB), lambda i: (0, i)),
        pl.BlockSpec((S, BB), lambda i: (0, i)),
    ]
    out_shape = [
        jax.ShapeDtypeStruct((S, S, b), jnp.float32),
        jax.ShapeDtypeStruct((S, b), jnp.float32),
        jax.ShapeDtypeStruct((S, b), jnp.int32),
        jax.ShapeDtypeStruct((S, b), jnp.float32),
    ]
    pa_t, p1_t, idx_t, sc_t = pl.pallas_call(
        _decode_block,
        grid=(nb,),
        in_specs=in_specs,
        out_specs=out_specs,
        out_shape=out_shape,
        compiler_params=pltpu.CompilerParams(
            dimension_semantics=("parallel",)),
    )(item_t, g_t, enc_W1, enc_b1.reshape(1, 32), enc_W2, enc_b2.reshape(1, 16),
      Wx, Wh, lstm_bias.reshape(1, 4 * H), dW1e, dW1h, dec_b1.reshape(1, 32),
      dec_W2, dec_b2.reshape(1, 16),
      dec_W3, dec_b3.reshape(1, 1))
    probs_all = jnp.transpose(pa_t, (2, 0, 1))
    probs_one = jnp.transpose(p1_t)
    idx = jnp.transpose(idx_t)
    scores = jnp.transpose(sc_t)
    return (probs_all, probs_one, idx, scores.reshape(-1, 1))


# speedup vs baseline: 4.5827x; 1.5988x over previous
import jax
import jax.numpy as jnp
from jax.experimental import pallas as pl
from jax.experimental.pallas import tpu as pltpu

S, F, H = 50, 128, 32
BB = 256


SP = 13


def _decode_block(item_ref, g_ref, encW1_ref, encb1_ref, encW2_ref, encb2_ref,
                  Wx_ref, Wh_ref, lb_ref, dW1e_ref, dW1h4_ref, db1t4_ref,
                  dW2d_ref, db2t4_ref, w3d_ref, db3_ref,
                  probs_all_ref, probs_one_ref, idx_ref, scores_ref):
    item2 = item_ref[...].reshape(S * BB, F)

    e1 = jnp.maximum(
        jnp.dot(item2, encW1_ref[...], preferred_element_type=jnp.float32)
        + encb1_ref[...], 0.0)
    e2 = jnp.maximum(
        jnp.dot(e1, encW2_ref[...], preferred_element_type=jnp.float32)
        + encb2_ref[...], 0.0)
    e2p = jnp.concatenate(
        [e2.reshape(S, BB, 16), jnp.zeros((4 * SP - S, BB, 16), jnp.float32)],
        axis=0).reshape(SP, 4, BB, 16)
    parts = [jnp.dot(e2p[:, q].reshape(SP * BB, 16), dW1e_ref[...],
                     preferred_element_type=jnp.float32) for q in range(4)]
    encprojW = (jnp.concatenate(parts, axis=1) + db1t4_ref[...]).reshape(SP, BB, 128)

    iota_sb = jax.lax.broadcasted_iota(jnp.int32, (S, BB), 0)
    iota_l = jax.lax.broadcasted_iota(jnp.int32, (S, BB, 1), 0)

    def body(k, carry):
        h, c, mask, dec_in, p1, ia, sa = carry
        gates = (jnp.dot(dec_in, Wx_ref[...], preferred_element_type=jnp.float32)
                 + jnp.dot(h, Wh_ref[...], preferred_element_type=jnp.float32)
                 + lb_ref[...])
        gi = gates[:, 0:H]
        gj = gates[:, H:2 * H]
        gf = gates[:, 2 * H:3 * H]
        go = gates[:, 3 * H:4 * H]
        c = c * jax.nn.sigmoid(gf + 1.0) + jax.nn.sigmoid(gi) * jnp.tanh(gj)
        h = jnp.tanh(c) * jax.nn.sigmoid(go)

        hp4 = jnp.dot(h, dW1h4_ref[...], preferred_element_type=jnp.float32)
        z1 = jnp.maximum(encprojW + hp4[None, :, :], 0.0)
        z2 = jnp.maximum(
            jnp.dot(z1.reshape(SP * BB, 128), dW2d_ref[...],
                    preferred_element_type=jnp.float32) + db2t4_ref[...], 0.0)
        lw = jnp.dot(z2, w3d_ref[...],
                     preferred_element_type=jnp.float32)
        lt = jnp.transpose(lw.reshape(SP, BB, 4), (0, 2, 1))
        logits = lt.reshape(4 * SP, BB)[:S]
        logits = logits + db3_ref[...]

        ml = (1.0 - mask) * (-1e9) + logits
        m = jnp.max(ml, axis=0, keepdims=True)
        e = jnp.exp(ml - m)
        p = e / jnp.sum(e, axis=0, keepdims=True)

        y = jnp.log(p + 1e-20) + g_ref[pl.ds(k, 1)][0]
        mx = jnp.max(y, axis=0, keepdims=True)
        idxv = jnp.min(jnp.where(y == mx, iota_sb, S), axis=0, keepdims=True)
        oh = (iota_sb == idxv).astype(jnp.float32)

        mask = mask - oh
        p1 = jnp.where(iota_sb == k, jnp.sum(p * oh, axis=0, keepdims=True), p1)
        ia = jnp.where(iota_sb == k, idxv, ia)
        sa = sa + oh * (1.0 - 0.03 * k.astype(jnp.float32))

        probs_all_ref[pl.ds(k, 1)] = p[None]

        idx_col = jnp.transpose(idxv)
        sel = iota_l == idx_col[None]
        dec_in = jnp.sum(jnp.where(sel, item_ref[...], 0.0), axis=0)
        return h, c, mask, dec_in, p1, ia, sa

    zf = jnp.zeros((BB, H), jnp.float32)
    carry = (zf, zf, jnp.ones((S, BB), jnp.float32), jnp.zeros((BB, F), jnp.float32),
             jnp.zeros((S, BB), jnp.float32), jnp.zeros((S, BB), jnp.int32),
             jnp.zeros((S, BB), jnp.float32))
    h, c, mask, dec_in, p1, ia, sa = jax.lax.fori_loop(0, S, body, carry)
    probs_one_ref[...] = p1
    idx_ref[...] = ia
    scores_ref[...] = sa


def kernel(item_input, enc_W1, enc_b1, enc_W2, enc_b2, lstm_kernel, lstm_bias,
           dec_W1, dec_b1, dec_W2, dec_b2, dec_W3, dec_b3):
    b = item_input.shape[0]
    nb = b // BB

    base = jax.random.key(42)
    g = jax.vmap(lambda k: jax.random.gumbel(jax.random.fold_in(base, k), (b, S),
                                             jnp.float32))(jnp.arange(S))
    g_t = jnp.transpose(g, (0, 2, 1))
    item_t = jnp.transpose(item_input, (1, 0, 2))

    Wx = lstm_kernel[:F]
    Wh = lstm_kernel[F:]
    dW1e = dec_W1[:16]
    dW1h4 = jnp.tile(dec_W1[16:], (1, 4))
    db1t4 = jnp.tile(dec_b1, 4).reshape(1, 128)
    dW2d = jnp.kron(jnp.eye(4, dtype=jnp.float32), dec_W2)
    db2t4 = jnp.tile(dec_b2, 4).reshape(1, 64)
    w3d = jnp.kron(jnp.eye(4, dtype=jnp.float32), dec_W3)

    full = lambda shp: pl.BlockSpec(shp, lambda i: tuple(0 for _ in shp))
    in_specs = [
        pl.BlockSpec((S, BB, F), lambda i: (0, i, 0)),
        pl.BlockSpec((S, S, BB), lambda i: (0, 0, i)),
        full((F, 32)), full((1, 32)), full((32, 16)), full((1, 16)),
        full((F, 4 * H)), full((H, 4 * H)), full((1, 4 * H)),
        full((16, 32)), full((H, 128)), full((1, 128)),
        full((128, 64)), full((1, 64)), full((64, 4)), full((1, 1)),
    ]
    out_specs = [
        pl.BlockSpec((S, S, BB), lambda i: (0, 0, i)),
        pl.BlockSpec((S, BB), lambda i: (0, i)),
        pl.BlockSpec((S, BB), lambda i: (0, i)),
        pl.BlockSpec((S, BB), lambda i: (0, i)),
    ]
    out_shape = [
        jax.ShapeDtypeStruct((S, S, b), jnp.float32),
        jax.ShapeDtypeStruct((S, b), jnp.float32),
        jax.ShapeDtypeStruct((S, b), jnp.int32),
        jax.ShapeDtypeStruct((S, b), jnp.float32),
    ]
    pa_t, p1_t, idx_t, sc_t = pl.pallas_call(
        _decode_block,
        grid=(nb,),
        in_specs=in_specs,
        out_specs=out_specs,
        out_shape=out_shape,
        compiler_params=pltpu.CompilerParams(
            dimension_semantics=("parallel",)),
    )(item_t, g_t, enc_W1, enc_b1.reshape(1, 32), enc_W2, enc_b2.reshape(1, 16),
      Wx, Wh, lstm_bias.reshape(1, 4 * H), dW1e, dW1h4, db1t4,
      dW2d, db2t4, w3d, dec_b3.reshape(1, 1))
    probs_all = jnp.transpose(pa_t, (2, 0, 1))
    probs_one = jnp.transpose(p1_t)
    idx = jnp.transpose(idx_t)
    scores = jnp.transpose(sc_t)
    return (probs_all, probs_one, idx, scores.reshape(-1, 1))


# speedup vs baseline: 4.6860x; 1.0225x over previous
import jax
import jax.numpy as jnp
from jax.experimental import pallas as pl
from jax.experimental.pallas import tpu as pltpu

S, F, H = 50, 128, 32
BB = 256


P = 8
SP = -(-S // P)


def _decode_block(item_ref, g_ref, encW1_ref, encb1_ref, encW2_ref, encb2_ref,
                  Wx_ref, Wh_ref, lb_ref, dW1e_ref, dW1h4_ref, db1t4_ref,
                  dW2d_ref, db2t4_ref, w3d_ref, db3_ref,
                  probs_all_ref, probs_one_ref, idx_ref, scores_ref):
    item2 = item_ref[...].reshape(S * BB, F)

    e1 = jnp.maximum(
        jnp.dot(item2, encW1_ref[...], preferred_element_type=jnp.float32)
        + encb1_ref[...], 0.0)
    e2 = jnp.maximum(
        jnp.dot(e1, encW2_ref[...], preferred_element_type=jnp.float32)
        + encb2_ref[...], 0.0)
    e2p = jnp.concatenate(
        [e2.reshape(S, BB, 16), jnp.zeros((P * SP - S, BB, 16), jnp.float32)],
        axis=0).reshape(SP, P, BB, 16)
    parts = [jnp.dot(e2p[:, q].reshape(SP * BB, 16), dW1e_ref[...],
                     preferred_element_type=jnp.float32) for q in range(P)]
    encprojW = (jnp.concatenate(parts, axis=1) + db1t4_ref[...]).reshape(SP, BB, P * 32)

    iota_sb = jax.lax.broadcasted_iota(jnp.int32, (S, BB), 0)
    iota_l = jax.lax.broadcasted_iota(jnp.int32, (S, BB, 1), 0)

    fg_one = jnp.where(
        (jax.lax.broadcasted_iota(jnp.int32, (1, 4 * H), 1) // H) == 2, 1.0, 0.0)

    def body(k, carry):
        h, c, mb, dec_in, p1, ia, sa = carry
        gates = (jnp.dot(dec_in, Wx_ref[...], preferred_element_type=jnp.float32)
                 + jnp.dot(h, Wh_ref[...], preferred_element_type=jnp.float32)
                 + lb_ref[...])
        sg = jax.nn.sigmoid(gates + fg_one)
        c = (c * sg[:, 2 * H:3 * H]
             + sg[:, 0:H] * jnp.tanh(gates[:, H:2 * H]))
        h = jnp.tanh(c) * sg[:, 3 * H:4 * H]

        hp4 = jnp.dot(h, dW1h4_ref[...], preferred_element_type=jnp.float32)
        z1 = jnp.maximum(encprojW + hp4[None, :, :], 0.0)
        z2 = jnp.maximum(
            jnp.dot(z1.reshape(SP * BB, P * 32), dW2d_ref[...],
                    preferred_element_type=jnp.float32) + db2t4_ref[...], 0.0)
        lw = jnp.dot(z2, w3d_ref[...],
                     preferred_element_type=jnp.float32)
        lt = jnp.transpose(lw.reshape(SP, BB, P), (0, 2, 1))
        logits = lt.reshape(P * SP, BB)[:S]
        logits = logits + db3_ref[...]

        ml = mb + logits
        m = jnp.max(ml, axis=0, keepdims=True)
        e = jnp.exp(ml - m)
        p = e / jnp.sum(e, axis=0, keepdims=True)

        y = jnp.log(p + 1e-20) + g_ref[pl.ds(k, 1)][0]
        mx = jnp.max(y, axis=0, keepdims=True)
        idxv = jnp.min(jnp.where(y == mx, iota_sb, S), axis=0, keepdims=True)
        oh = (iota_sb == idxv).astype(jnp.float32)

        mb = mb + oh * (-1e9)
        p1 = jnp.where(iota_sb == k, jnp.sum(p * oh, axis=0, keepdims=True), p1)
        ia = jnp.where(iota_sb == k, idxv, ia)
        sa = sa + oh * (1.0 - 0.03 * k.astype(jnp.float32))

        probs_all_ref[pl.ds(k, 1)] = p[None]

        idx_col = jnp.transpose(idxv)
        sel = iota_l == idx_col[None]
        dec_in = jnp.sum(jnp.where(sel, item_ref[...], 0.0), axis=0)
        return h, c, mb, dec_in, p1, ia, sa

    zf = jnp.zeros((BB, H), jnp.float32)
    carry = (zf, zf, jnp.zeros((S, BB), jnp.float32), jnp.zeros((BB, F), jnp.float32),
             jnp.zeros((S, BB), jnp.float32), jnp.zeros((S, BB), jnp.int32),
             jnp.zeros((S, BB), jnp.float32))
    h, c, mb, dec_in, p1, ia, sa = jax.lax.fori_loop(0, S, body, carry)
    probs_one_ref[...] = p1
    idx_ref[...] = ia
    scores_ref[...] = sa


def kernel(item_input, enc_W1, enc_b1, enc_W2, enc_b2, lstm_kernel, lstm_bias,
           dec_W1, dec_b1, dec_W2, dec_b2, dec_W3, dec_b3):
    b = item_input.shape[0]
    nb = b // BB

    base = jax.random.key(42)
    g = jax.vmap(lambda k: jax.random.gumbel(jax.random.fold_in(base, k), (b, S),
                                             jnp.float32))(jnp.arange(S))
    g_t = jnp.transpose(g, (0, 2, 1))
    item_t = jnp.transpose(item_input, (1, 0, 2))

    Wx = lstm_kernel[:F]
    Wh = lstm_kernel[F:]
    dW1e = dec_W1[:16]
    dW1h4 = jnp.tile(dec_W1[16:], (1, P))
    db1t4 = jnp.tile(dec_b1, P).reshape(1, P * 32)
    dW2d = jnp.kron(jnp.eye(P, dtype=jnp.float32), dec_W2)
    db2t4 = jnp.tile(dec_b2, P).reshape(1, P * 16)
    w3d = jnp.kron(jnp.eye(P, dtype=jnp.float32), dec_W3)

    full = lambda shp: pl.BlockSpec(shp, lambda i: tuple(0 for _ in shp))
    in_specs = [
        pl.BlockSpec((S, BB, F), lambda i: (0, i, 0)),
        pl.BlockSpec((S, S, BB), lambda i: (0, 0, i)),
        full((F, 32)), full((1, 32)), full((32, 16)), full((1, 16)),
        full((F, 4 * H)), full((H, 4 * H)), full((1, 4 * H)),
        full((16, 32)), full((H, P * 32)), full((1, P * 32)),
        full((P * 32, P * 16)), full((1, P * 16)), full((P * 16, P)), full((1, 1)),
    ]
    out_specs = [
        pl.BlockSpec((S, S, BB), lambda i: (0, 0, i)),
        pl.BlockSpec((S, BB), lambda i: (0, i)),
        pl.BlockSpec((S, BB), lambda i: (0, i)),
        pl.BlockSpec((S, BB), lambda i: (0, i)),
    ]
    out_shape = [
        jax.ShapeDtypeStruct((S, S, b), jnp.float32),
        jax.ShapeDtypeStruct((S, b), jnp.float32),
        jax.ShapeDtypeStruct((S, b), jnp.int32),
        jax.ShapeDtypeStruct((S, b), jnp.float32),
    ]
    pa_t, p1_t, idx_t, sc_t = pl.pallas_call(
        _decode_block,
        grid=(nb,),
        in_specs=in_specs,
        out_specs=out_specs,
        out_shape=out_shape,
        compiler_params=pltpu.CompilerParams(
            dimension_semantics=("parallel",)),
    )(item_t, g_t, enc_W1, enc_b1.reshape(1, 32), enc_W2, enc_b2.reshape(1, 16),
      Wx, Wh, lstm_bias.reshape(1, 4 * H), dW1e, dW1h4, db1t4,
      dW2d, db2t4, w3d, dec_b3.reshape(1, 1))
    probs_all = jnp.transpose(pa_t, (2, 0, 1))
    probs_one = jnp.transpose(p1_t)
    idx = jnp.transpose(idx_t)
    scores = jnp.transpose(sc_t)
    return (probs_all, probs_one, idx, scores.reshape(-1, 1))


# speedup vs baseline: 4.8554x; 1.0361x over previous
import jax
import jax.numpy as jnp
from jax.experimental import pallas as pl
from jax.experimental.pallas import tpu as pltpu

S, F, H = 50, 128, 32
BB = 256


P = 8
SP = -(-S // P)


def _decode_block(item_ref, g_ref, encW1_ref, encb1_ref, encW2_ref, encb2_ref,
                  Wx_ref, Wh_ref, lb_ref, dW1e_ref, dW1h4_ref, db1t4_ref,
                  dW2d_ref, db2t4_ref, w3d_ref, db3_ref,
                  probs_all_ref, probs_one_ref, idx_ref, scores_ref):
    item2 = item_ref[...].reshape(S * BB, F)

    e1 = jnp.maximum(
        jnp.dot(item2, encW1_ref[...], preferred_element_type=jnp.float32)
        + encb1_ref[...], 0.0)
    e2 = jnp.maximum(
        jnp.dot(e1, encW2_ref[...], preferred_element_type=jnp.float32)
        + encb2_ref[...], 0.0)
    e2p = jnp.concatenate(
        [e2.reshape(S, BB, 16), jnp.zeros((P * SP - S, BB, 16), jnp.float32)],
        axis=0).reshape(SP, P, BB, 16)
    parts = [jnp.dot(e2p[:, q].reshape(SP * BB, 16), dW1e_ref[...],
                     preferred_element_type=jnp.float32) for q in range(P)]
    encprojW = (jnp.concatenate(parts, axis=1) + db1t4_ref[...]).reshape(SP, BB, P * 32)

    itemx = jnp.dot(item2, Wx_ref[...],
                    preferred_element_type=jnp.float32).reshape(S, BB, F)

    iota_sb = jax.lax.broadcasted_iota(jnp.int32, (S, BB), 0)
    iota_l = jax.lax.broadcasted_iota(jnp.int32, (S, BB, 1), 0)

    fg_one = jnp.where(
        (jax.lax.broadcasted_iota(jnp.int32, (1, 4 * H), 1) // H) == 2, 1.0, 0.0)

    def body(k, carry):
        h, c, mb, gx, p1, ia, sa = carry
        gates = (gx
                 + jnp.dot(h, Wh_ref[...], preferred_element_type=jnp.float32)
                 + lb_ref[...])
        sg = jax.nn.sigmoid(gates + fg_one)
        c = (c * sg[:, 2 * H:3 * H]
             + sg[:, 0:H] * jnp.tanh(gates[:, H:2 * H]))
        h = jnp.tanh(c) * sg[:, 3 * H:4 * H]

        hp4 = jnp.dot(h, dW1h4_ref[...], preferred_element_type=jnp.float32)
        z1 = jnp.maximum(encprojW + hp4[None, :, :], 0.0)
        z2 = jnp.maximum(
            jnp.dot(z1.reshape(SP * BB, P * 32), dW2d_ref[...],
                    preferred_element_type=jnp.float32) + db2t4_ref[...], 0.0)
        lw = jnp.dot(z2, w3d_ref[...],
                     preferred_element_type=jnp.float32)
        lt = jnp.transpose(lw.reshape(SP, BB, P), (0, 2, 1))
        logits = lt.reshape(P * SP, BB)[:S]
        logits = logits + db3_ref[...]

        ml = mb + logits
        m = jnp.max(ml, axis=0, keepdims=True)
        e = jnp.exp(ml - m)
        p = e / jnp.sum(e, axis=0, keepdims=True)

        y = jnp.log(p + 1e-20) + g_ref[pl.ds(k, 1)][0]
        mx = jnp.max(y, axis=0, keepdims=True)
        idxv = jnp.min(jnp.where(y == mx, iota_sb, S), axis=0, keepdims=True)
        oh = (iota_sb == idxv).astype(jnp.float32)

        mb = mb + oh * (-1e9)
        p1 = jnp.where(iota_sb == k, jnp.sum(p * oh, axis=0, keepdims=True), p1)
        ia = jnp.where(iota_sb == k, idxv, ia)
        sa = sa + oh * (1.0 - 0.03 * k.astype(jnp.float32))

        probs_all_ref[pl.ds(k, 1)] = p[None]

        idx_col = jnp.transpose(idxv)
        sel = iota_l == idx_col[None]
        gx = jnp.sum(jnp.where(sel, itemx, 0.0), axis=0)
        return h, c, mb, gx, p1, ia, sa

    zf = jnp.zeros((BB, H), jnp.float32)
    carry = (zf, zf, jnp.zeros((S, BB), jnp.float32), jnp.zeros((BB, F), jnp.float32),
             jnp.zeros((S, BB), jnp.float32), jnp.zeros((S, BB), jnp.int32),
             jnp.zeros((S, BB), jnp.float32))
    h, c, mb, gx, p1, ia, sa = jax.lax.fori_loop(0, S, body, carry, unroll=2)
    probs_one_ref[...] = p1
    idx_ref[...] = ia
    scores_ref[...] = sa


def kernel(item_input, enc_W1, enc_b1, enc_W2, enc_b2, lstm_kernel, lstm_bias,
           dec_W1, dec_b1, dec_W2, dec_b2, dec_W3, dec_b3):
    b = item_input.shape[0]
    nb = b // BB

    base = jax.random.key(42)
    g = jax.vmap(lambda k: jax.random.gumbel(jax.random.fold_in(base, k), (b, S),
                                             jnp.float32))(jnp.arange(S))
    g_t = jnp.transpose(g, (0, 2, 1))
    item_t = jnp.transpose(item_input, (1, 0, 2))

    Wx = lstm_kernel[:F]
    Wh = lstm_kernel[F:]
    dW1e = dec_W1[:16]
    dW1h4 = jnp.tile(dec_W1[16:], (1, P))
    db1t4 = jnp.tile(dec_b1, P).reshape(1, P * 32)
    dW2d = jnp.kron(jnp.eye(P, dtype=jnp.float32), dec_W2)
    db2t4 = jnp.tile(dec_b2, P).reshape(1, P * 16)
    w3d = jnp.kron(jnp.eye(P, dtype=jnp.float32), dec_W3)

    full = lambda shp: pl.BlockSpec(shp, lambda i: tuple(0 for _ in shp))
    in_specs = [
        pl.BlockSpec((S, BB, F), lambda i: (0, i, 0)),
        pl.BlockSpec((S, S, BB), lambda i: (0, 0, i)),
        full((F, 32)), full((1, 32)), full((32, 16)), full((1, 16)),
        full((F, 4 * H)), full((H, 4 * H)), full((1, 4 * H)),
        full((16, 32)), full((H, P * 32)), full((1, P * 32)),
        full((P * 32, P * 16)), full((1, P * 16)), full((P * 16, P)), full((1, 1)),
    ]
    out_specs = [
        pl.BlockSpec((S, S, BB), lambda i: (0, 0, i)),
        pl.BlockSpec((S, BB), lambda i: (0, i)),
        pl.BlockSpec((S, BB), lambda i: (0, i)),
        pl.BlockSpec((S, BB), lambda i: (0, i)),
    ]
    out_shape = [
        jax.ShapeDtypeStruct((S, S, b), jnp.float32),
        jax.ShapeDtypeStruct((S, b), jnp.float32),
        jax.ShapeDtypeStruct((S, b), jnp.int32),
        jax.ShapeDtypeStruct((S, b), jnp.float32),
    ]
    pa_t, p1_t, idx_t, sc_t = pl.pallas_call(
        _decode_block,
        grid=(nb,),
        in_specs=in_specs,
        out_specs=out_specs,
        out_shape=out_shape,
        compiler_params=pltpu.CompilerParams(
            dimension_semantics=("parallel",)),
    )(item_t, g_t, enc_W1, enc_b1.reshape(1, 32), enc_W2, enc_b2.reshape(1, 16),
      Wx, Wh, lstm_bias.reshape(1, 4 * H), dW1e, dW1h4, db1t4,
      dW2d, db2t4, w3d, dec_b3.reshape(1, 1))
    probs_all = jnp.transpose(pa_t, (2, 0, 1))
    probs_one = jnp.transpose(p1_t)
    idx = jnp.transpose(idx_t)
    scores = jnp.transpose(sc_t)
    return (probs_all, probs_one, idx, scores.reshape(-1, 1))


# speedup vs baseline: 5.0016x; 1.0301x over previous
import jax
import jax.numpy as jnp
from jax.experimental import pallas as pl
from jax.experimental.pallas import tpu as pltpu

S, F, H = 50, 128, 32
BB = 256


P = 8
SP = -(-S // P)


def _decode_block(item_ref, g_ref, encW1_ref, encb1_ref, encW2_ref, encb2_ref,
                  Wx_ref, Wh_ref, lb_ref, dW1e_ref, dW1h4_ref, db1t4_ref,
                  dW2d_ref, db2t4_ref, w3d_ref, db3_ref,
                  probs_all_ref, probs_one_ref, idx_ref, scores_ref):
    item2 = item_ref[...].reshape(S * BB, F)

    e1 = jnp.maximum(
        jnp.dot(item2, encW1_ref[...], preferred_element_type=jnp.float32)
        + encb1_ref[...], 0.0)
    e2 = jnp.maximum(
        jnp.dot(e1, encW2_ref[...], preferred_element_type=jnp.float32)
        + encb2_ref[...], 0.0)
    e2p = jnp.concatenate(
        [e2.reshape(S, BB, 16), jnp.zeros((P * SP - S, BB, 16), jnp.float32)],
        axis=0).reshape(SP, P, BB, 16)
    parts = [jnp.dot(e2p[:, q].reshape(SP * BB, 16), dW1e_ref[...],
                     preferred_element_type=jnp.float32) for q in range(P)]
    encprojW = (jnp.concatenate(parts, axis=1) + db1t4_ref[...]).reshape(SP, BB, P * 32)

    itemx = jnp.dot(item2, Wx_ref[...],
                    preferred_element_type=jnp.float32).reshape(S, BB, F)

    iota_sb = jax.lax.broadcasted_iota(jnp.int32, (S, BB), 0)
    iota_l = jax.lax.broadcasted_iota(jnp.int32, (S, BB, 1), 0)

    fg_one = jnp.where(
        (jax.lax.broadcasted_iota(jnp.int32, (1, 4 * H), 1) // H) == 2, 1.0, 0.0)

    def body(k, carry):
        h, c, mb, gx, p1, ia, sa = carry
        gates = (gx
                 + jnp.dot(h, Wh_ref[...], preferred_element_type=jnp.float32)
                 + lb_ref[...])
        sg = jax.nn.sigmoid(gates + fg_one)
        c = (c * sg[:, 2 * H:3 * H]
             + sg[:, 0:H] * jnp.tanh(gates[:, H:2 * H]))
        h = jnp.tanh(c) * sg[:, 3 * H:4 * H]

        hp4 = jnp.dot(h, dW1h4_ref[...], preferred_element_type=jnp.float32)
        z1 = jnp.maximum(encprojW + hp4[None, :, :], 0.0)
        z2 = jnp.maximum(
            jnp.dot(z1.reshape(SP * BB, P * 32), dW2d_ref[...],
                    preferred_element_type=jnp.float32) + db2t4_ref[...], 0.0)
        lw = jnp.dot(z2, w3d_ref[...],
                     preferred_element_type=jnp.float32)
        lt = jnp.transpose(lw.reshape(SP, BB, P), (0, 2, 1))
        logits = lt.reshape(P * SP, BB)[:S]
        logits = logits + db3_ref[...]

        ml = mb + logits
        m = jnp.max(ml, axis=0, keepdims=True)
        e = jnp.exp(ml - m)
        p = e / jnp.sum(e, axis=0, keepdims=True)

        y = jnp.log(p + 1e-20) + g_ref[pl.ds(k, 1)][0]
        mx = jnp.max(y, axis=0, keepdims=True)
        idxv = jnp.min(jnp.where(y == mx, iota_sb, S), axis=0, keepdims=True)
        oh = (iota_sb == idxv).astype(jnp.float32)

        mb = mb + oh * (-1e9)
        p1 = jnp.where(iota_sb == k, jnp.sum(p * oh, axis=0, keepdims=True), p1)
        ia = jnp.where(iota_sb == k, idxv, ia)
        sa = sa + oh * (1.0 - 0.03 * k.astype(jnp.float32))

        probs_all_ref[pl.ds(k, 1)] = p[None]

        idx_col = jnp.transpose(idxv)
        sel = iota_l == idx_col[None]
        gx = jnp.sum(jnp.where(sel, itemx, 0.0), axis=0)
        return h, c, mb, gx, p1, ia, sa

    zf = jnp.zeros((BB, H), jnp.float32)
    carry = (zf, zf, jnp.zeros((S, BB), jnp.float32), jnp.zeros((BB, F), jnp.float32),
             jnp.zeros((S, BB), jnp.float32), jnp.zeros((S, BB), jnp.int32),
             jnp.zeros((S, BB), jnp.float32))
    h, c, mb, gx, p1, ia, sa = jax.lax.fori_loop(0, S, body, carry, unroll=4)
    probs_one_ref[...] = p1
    idx_ref[...] = ia
    scores_ref[...] = sa


def kernel(item_input, enc_W1, enc_b1, enc_W2, enc_b2, lstm_kernel, lstm_bias,
           dec_W1, dec_b1, dec_W2, dec_b2, dec_W3, dec_b3):
    b = item_input.shape[0]
    nb = b // BB

    base = jax.random.key(42)
    g = jax.vmap(lambda k: jax.random.gumbel(jax.random.fold_in(base, k), (b, S),
                                             jnp.float32))(jnp.arange(S))
    g_t = jnp.transpose(g, (0, 2, 1))
    item_t = jnp.transpose(item_input, (1, 0, 2))

    Wx = lstm_kernel[:F]
    Wh = lstm_kernel[F:]
    dW1e = dec_W1[:16]
    dW1h4 = jnp.tile(dec_W1[16:], (1, P))
    db1t4 = jnp.tile(dec_b1, P).reshape(1, P * 32)
    dW2d = jnp.kron(jnp.eye(P, dtype=jnp.float32), dec_W2)
    db2t4 = jnp.tile(dec_b2, P).reshape(1, P * 16)
    w3d = jnp.kron(jnp.eye(P, dtype=jnp.float32), dec_W3)

    full = lambda shp: pl.BlockSpec(shp, lambda i: tuple(0 for _ in shp))
    in_specs = [
        pl.BlockSpec((S, BB, F), lambda i: (0, i, 0)),
        pl.BlockSpec((S, S, BB), lambda i: (0, 0, i)),
        full((F, 32)), full((1, 32)), full((32, 16)), full((1, 16)),
        full((F, 4 * H)), full((H, 4 * H)), full((1, 4 * H)),
        full((16, 32)), full((H, P * 32)), full((1, P * 32)),
        full((P * 32, P * 16)), full((1, P * 16)), full((P * 16, P)), full((1, 1)),
    ]
    out_specs = [
        pl.BlockSpec((S, S, BB), lambda i: (0, 0, i)),
        pl.BlockSpec((S, BB), lambda i: (0, i)),
        pl.BlockSpec((S, BB), lambda i: (0, i)),
        pl.BlockSpec((S, BB), lambda i: (0, i)),
    ]
    out_shape = [
        jax.ShapeDtypeStruct((S, S, b), jnp.float32),
        jax.ShapeDtypeStruct((S, b), jnp.float32),
        jax.ShapeDtypeStruct((S, b), jnp.int32),
        jax.ShapeDtypeStruct((S, b), jnp.float32),
    ]
    pa_t, p1_t, idx_t, sc_t = pl.pallas_call(
        _decode_block,
        grid=(nb,),
        in_specs=in_specs,
        out_specs=out_specs,
        out_shape=out_shape,
        compiler_params=pltpu.CompilerParams(
            dimension_semantics=("parallel",)),
    )(item_t, g_t, enc_W1, enc_b1.reshape(1, 32), enc_W2, enc_b2.reshape(1, 16),
      Wx, Wh, lstm_bias.reshape(1, 4 * H), dW1e, dW1h4, db1t4,
      dW2d, db2t4, w3d, dec_b3.reshape(1, 1))
    probs_all = jnp.transpose(pa_t, (2, 0, 1))
    probs_one = jnp.transpose(p1_t)
    idx = jnp.transpose(idx_t)
    scores = jnp.transpose(sc_t)
    return (probs_all, probs_one, idx, scores.reshape(-1, 1))
